# baseline (speedup 1.0000x reference)
import functools

import jax
import jax.numpy as jnp
from jax import lax
from jax.experimental import pallas as pl
from jax.experimental.pallas import tpu as pltpu
from jax.experimental.pallas import tpu_sc as plsc

_NC = 2
_NS = 16
_NW = _NC * _NS

_G = 128


@functools.partial(jax.jit, static_argnames=("K",))
def _sc_gather(idx2d, table, *, K):
    R = idx2d.shape[0]
    D = table.shape[1]
    C = K * _G
    rows_per_w = R // _NW
    b_per_w = rows_per_w * _G
    n_chunks = rows_per_w // K
    B = R * _G

    mesh = plsc.VectorSubcoreMesh(core_axis_name="c", subcore_axis_name="s")

    @functools.partial(
        pl.kernel,
        out_type=jax.ShapeDtypeStruct((B, D), jnp.float32),
        mesh=mesh,
        scratch_types=[
            pltpu.VMEM((K, _G), jnp.int32),
            pltpu.VMEM((C, D), jnp.float32),
            pltpu.SemaphoreType.DMA,
        ],
        compiler_params=pltpu.CompilerParams(use_tc_tiling_on_sc=False),
    )
    def k(idx_hbm, table_hbm, out_hbm, idx_v, rows_v, sem):
        wid = lax.axis_index("s") * _NC + lax.axis_index("c")
        row_base = wid * rows_per_w
        idx_base = wid * b_per_w

        def body(i, carry):
            pltpu.sync_copy(idx_hbm.at[pl.ds(row_base + i * K, K)], idx_v)
            copies = []
            for j in range(K):
                copies.append(pltpu.async_copy(
                    table_hbm.at[idx_v.at[j]],
                    rows_v.at[pl.ds(j * _G, _G)],
                    sem,
                ))
            for c in copies:
                c.wait()
            pltpu.sync_copy(rows_v, out_hbm.at[pl.ds(idx_base + i * C, C)])
            return carry

        lax.fori_loop(0, n_chunks, body, 0)

    return k(idx2d, table)


def kernel(x, table):
    B = x.size
    idx2d = x.reshape(B // _G, _G).astype(jnp.int32)
    out = _sc_gather(idx2d, table, K=8)
    return out.reshape(x.shape + (table.shape[1],))


# speedup vs baseline: 1.0141x; 1.0141x over previous
import functools

import jax
import jax.numpy as jnp
from jax import lax
from jax.experimental import pallas as pl
from jax.experimental.pallas import tpu as pltpu
from jax.experimental.pallas import tpu_sc as plsc

_NC = 2
_NS = 16
_NW = _NC * _NS

_G = 128


@functools.partial(jax.jit, static_argnames=("K",))
def _sc_gather(idx2d, table, *, K):
    R = idx2d.shape[0]
    D = table.shape[1]
    C = K * _G
    rows_per_w = R // _NW
    b_per_w = rows_per_w * _G
    n_chunks = rows_per_w // K
    assert rows_per_w % K == 0 and n_chunks % 2 == 0 and R % _NW == 0
    B = R * _G

    mesh = plsc.VectorSubcoreMesh(core_axis_name="c", subcore_axis_name="s")

    @functools.partial(
        pl.kernel,
        out_type=jax.ShapeDtypeStruct((B, D), jnp.float32),
        mesh=mesh,
        scratch_types=[
            pltpu.VMEM((2, K, _G), jnp.int32),
            pltpu.VMEM((2, C, D), jnp.float32),
            pltpu.SemaphoreType.DMA((2,)),
            pltpu.SemaphoreType.DMA((2,)),
            pltpu.SemaphoreType.DMA((2,)),
        ],
        compiler_params=pltpu.CompilerParams(use_tc_tiling_on_sc=False),
    )
    def k(idx_hbm, table_hbm, out_hbm, idx_v, rows_v, sem_i, sem_g, sem_s):
        wid = lax.axis_index("s") * _NC + lax.axis_index("c")
        row_base = wid * rows_per_w
        idx_base = wid * b_per_w

        def idx_copy(step, b):
            return pltpu.make_async_copy(
                idx_hbm.at[pl.ds(row_base + step * K, K)],
                idx_v.at[b], sem_i.at[b])

        def gather_copy(b, j):
            return pltpu.make_async_copy(
                table_hbm.at[idx_v.at[b, j]],
                rows_v.at[b, pl.ds(j * _G, _G)], sem_g.at[b])

        def store_copy(step, b):
            return pltpu.make_async_copy(
                rows_v.at[b],
                out_hbm.at[pl.ds(idx_base + step * C, C)], sem_s.at[b])

        idx_copy(0, 0).start()
        idx_copy(1, 1).start()

        def body(t, carry):
            g = t * 2
            for b in range(2):
                step = g + b
                idx_copy(step, b).wait()

                @pl.when(step >= 2)
                def _():
                    store_copy(step - 2, b).wait()

                for j in range(K):
                    gather_copy(b, j).start()
                for j in range(K):
                    gather_copy(b, j).wait()

                @pl.when(step + 2 < n_chunks)
                def _():
                    idx_copy(step + 2, b).start()

                store_copy(step, b).start()
            return carry

        lax.fori_loop(0, n_chunks // 2, body, 0)
        store_copy(n_chunks - 2, 0).wait()
        store_copy(n_chunks - 1, 1).wait()

    return k(idx2d, table)


def kernel(x, table):
    B = x.size
    idx2d = x.reshape(B // _G, _G).astype(jnp.int32)
    out = _sc_gather(idx2d, table, K=5)
    return out.reshape(x.shape + (table.shape[1],))


# speedup vs baseline: 1.0163x; 1.0021x over previous
import functools

import jax
import jax.numpy as jnp
from jax import lax
from jax.experimental import pallas as pl
from jax.experimental.pallas import tpu as pltpu
from jax.experimental.pallas import tpu_sc as plsc

_NC = 2
_NS = 16
_NW = _NC * _NS

_RC = 4


@jax.jit
def _sc_gather(x, table):
    N, J = x.shape
    D = table.shape[1]
    rows_per_w = N // _NW
    n_chunks = rows_per_w // _RC
    assert N % _NW == 0 and rows_per_w % _RC == 0 and n_chunks % 2 == 0
    pieces = []
    off = 0
    while off < J:
        n = min(128, J - off)
        pieces.append((off, n))
        off += n
    assert all(o % 8 == 0 and n % 8 == 0 for o, n in pieces)

    mesh = plsc.VectorSubcoreMesh(core_axis_name="c", subcore_axis_name="s")

    @functools.partial(
        pl.kernel,
        out_type=jax.ShapeDtypeStruct((N, J, D), jnp.float32),
        mesh=mesh,
        scratch_types=[
            pltpu.VMEM((2, _RC, J), jnp.int32),
            pltpu.VMEM((2, _RC, J, D), jnp.float32),
            pltpu.SemaphoreType.DMA((2,)),
            pltpu.SemaphoreType.DMA((2,)),
            pltpu.SemaphoreType.DMA((2,)),
        ],
        compiler_params=pltpu.CompilerParams(use_tc_tiling_on_sc=False),
    )
    def k(x_hbm, table_hbm, out_hbm, idx_v, rows_v, sem_i, sem_g, sem_s):
        wid = lax.axis_index("s") * _NC + lax.axis_index("c")
        row_base = wid * rows_per_w

        def idx_copy(step, b):
            return pltpu.make_async_copy(
                x_hbm.at[pl.ds(row_base + step * _RC, _RC)],
                idx_v.at[b], sem_i.at[b])

        def gather_copy(b, r, o, n):
            return pltpu.make_async_copy(
                table_hbm.at[idx_v.at[b, r, pl.ds(o, n)]],
                rows_v.at[b, r, pl.ds(o, n)], sem_g.at[b])

        def store_copy(step, b):
            return pltpu.make_async_copy(
                rows_v.at[b],
                out_hbm.at[pl.ds(row_base + step * _RC, _RC)], sem_s.at[b])

        idx_copy(0, 0).start()
        idx_copy(1, 1).start()

        def body(t, carry):
            g = t * 2
            for b in range(2):
                step = g + b
                idx_copy(step, b).wait()

                @pl.when(step >= 2)
                def _():
                    store_copy(step - 2, b).wait()

                for r in range(_RC):
                    for o, n in pieces:
                        gather_copy(b, r, o, n).start()
                for r in range(_RC):
                    for o, n in pieces:
                        gather_copy(b, r, o, n).wait()

                @pl.when(step + 2 < n_chunks)
                def _():
                    idx_copy(step + 2, b).start()

                store_copy(step, b).start()
            return carry

        lax.fori_loop(0, n_chunks // 2, body, 0)
        store_copy(n_chunks - 2, 0).wait()
        store_copy(n_chunks - 1, 1).wait()

    return k(x, table)


def kernel(x, table):
    return _sc_gather(x.astype(jnp.int32), table)


# speedup vs baseline: 1.3514x; 1.3298x over previous
import functools

import jax
import jax.numpy as jnp
from jax import lax
from jax.experimental import pallas as pl
from jax.experimental.pallas import tpu as pltpu
from jax.experimental.pallas import tpu_sc as plsc

_NC = 2
_NS = 16
_NW = _NC * _NS

_RC = 4


@jax.jit
def _sc_gather(x, table):
    N, J = x.shape
    D = table.shape[1]
    C = _RC * J
    rows_per_w = N // _NW
    n_chunks = rows_per_w // _RC
    assert N % _NW == 0 and rows_per_w % _RC == 0 and n_chunks % 2 == 0
    pieces = []
    off = 0
    while off < J:
        n = min(128, J - off)
        pieces.append((off, n))
        off += n
    assert all(o % 8 == 0 and n % 8 == 0 for o, n in pieces)

    mesh = plsc.VectorSubcoreMesh(core_axis_name="c", subcore_axis_name="s")

    @functools.partial(
        pl.kernel,
        out_type=jax.ShapeDtypeStruct((N * J, 2 * D), jnp.float32),
        mesh=mesh,
        scratch_types=[
            pltpu.VMEM((2, _RC, J), jnp.int32),
            pltpu.VMEM((2, C, D), jnp.float32),
            pltpu.SemaphoreType.DMA((2,)),
            pltpu.SemaphoreType.DMA((2,)),
            pltpu.SemaphoreType.DMA((2,)),
        ],
        compiler_params=pltpu.CompilerParams(use_tc_tiling_on_sc=False),
    )
    def k(x_hbm, table_hbm, out_hbm, idx_v, rows_v, sem_i, sem_g, sem_s):
        wid = lax.axis_index("s") * _NC + lax.axis_index("c")
        row_base = wid * rows_per_w

        def idx_copy(step, b):
            return pltpu.make_async_copy(
                x_hbm.at[pl.ds(row_base + step * _RC, _RC)],
                idx_v.at[b], sem_i.at[b])

        def gather_copy(b, r, o, n):
            return pltpu.make_async_copy(
                table_hbm.at[idx_v.at[b, r, pl.ds(o, n)]],
                rows_v.at[b, pl.ds(r * J + o, n)], sem_g.at[b])

        def store_copy(step, b):
            return pltpu.make_async_copy(
                rows_v.at[b],
                out_hbm.at[pl.ds((row_base + step * _RC) * J, C),
                           pl.ds(0, D)],
                sem_s.at[b])

        idx_copy(0, 0).start()
        idx_copy(1, 1).start()

        def body(t, carry):
            g = t * 2
            for b in range(2):
                step = g + b
                idx_copy(step, b).wait()

                @pl.when(step >= 2)
                def _():
                    store_copy(step - 2, b).wait()

                for r in range(_RC):
                    for o, n in pieces:
                        gather_copy(b, r, o, n).start()
                for r in range(_RC):
                    for o, n in pieces:
                        gather_copy(b, r, o, n).wait()

                @pl.when(step + 2 < n_chunks)
                def _():
                    idx_copy(step + 2, b).start()

                store_copy(step, b).start()
            return carry

        lax.fori_loop(0, n_chunks // 2, body, 0)
        store_copy(n_chunks - 2, 0).wait()
        store_copy(n_chunks - 1, 1).wait()

    return k(x, table)


def kernel(x, table):
    out128 = _sc_gather(x.astype(jnp.int32), table)
    return out128[:, :64].reshape(x.shape + (table.shape[1],))


# speedup vs baseline: 1.3542x; 1.0021x over previous
import functools

import jax
import jax.numpy as jnp
from jax import lax
from jax.experimental import pallas as pl
from jax.experimental.pallas import tpu as pltpu
from jax.experimental.pallas import tpu_sc as plsc

_NC = 2
_NS = 16
_NW = _NC * _NS

_RC = 4


@jax.jit
def _sc_gather(x, table):
    N, J = x.shape
    D = table.shape[1]
    C = _RC * J
    rows_per_w = N // _NW
    n_chunks = rows_per_w // _RC
    assert N % _NW == 0 and rows_per_w % _RC == 0 and n_chunks % 2 == 0
    pieces = []
    off = 0
    while off < J:
        n = min(128, J - off)
        pieces.append((off, n))
        off += n
    assert all(o % 8 == 0 and n % 8 == 0 for o, n in pieces)

    mesh = plsc.VectorSubcoreMesh(core_axis_name="c", subcore_axis_name="s")

    @functools.partial(
        pl.kernel,
        out_type=jax.ShapeDtypeStruct((N * J, 2 * D), jnp.float32),
        mesh=mesh,
        scratch_types=[
            pltpu.VMEM((2, _RC, J), jnp.int32),
            pltpu.VMEM((2, C, D), jnp.float32),
            pltpu.SemaphoreType.DMA((2,)),
            pltpu.SemaphoreType.DMA((2,)),
            pltpu.SemaphoreType.DMA((2,)),
        ],
        compiler_params=pltpu.CompilerParams(use_tc_tiling_on_sc=False),
    )
    def k(x_hbm, table_hbm, out_hbm, idx_v, rows_v, sem_i, sem_g, sem_s):
        wid = lax.axis_index("s") * _NC + lax.axis_index("c")
        row_base = wid * rows_per_w

        def idx_copy(step, b):
            return pltpu.make_async_copy(
                x_hbm.at[pl.ds(row_base + step * _RC, _RC)],
                idx_v.at[b], sem_i.at[b])

        def gather_copy(b, r, o, n):
            return pltpu.make_async_copy(
                table_hbm.at[idx_v.at[b, r, pl.ds(o, n)]],
                rows_v.at[b, pl.ds(r * J + o, n)], sem_g.at[b])

        def store_copy(step, b):
            return pltpu.make_async_copy(
                rows_v.at[b],
                out_hbm.at[pl.ds((row_base + step * _RC) * J, C),
                           pl.ds(0, D)],
                sem_s.at[b])

        def fire_gathers(b):
            for r in range(_RC):
                for o, n in pieces:
                    gather_copy(b, r, o, n).start()

        def wait_gathers(b):
            for r in range(_RC):
                for o, n in pieces:
                    gather_copy(b, r, o, n).wait()

        idx_copy(0, 0).start()
        idx_copy(1, 1).start()
        idx_copy(0, 0).wait()
        fire_gathers(0)

        def body(t, carry):
            g = t * 2
            for b in range(2):
                step = g + b
                nb = 1 - b

                @pl.when(step + 1 < n_chunks)
                def _():
                    idx_copy(step + 1, nb).wait()

                    @pl.when(step >= 1)
                    def _():
                        store_copy(step - 1, nb).wait()

                    fire_gathers(nb)

                wait_gathers(b)
                store_copy(step, b).start()

                @pl.when(step + 2 < n_chunks)
                def _():
                    idx_copy(step + 2, b).start()
            return carry

        lax.fori_loop(0, n_chunks // 2, body, 0)
        store_copy(n_chunks - 2, 0).wait()
        store_copy(n_chunks - 1, 1).wait()

    return k(x, table)


def kernel(x, table):
    out128 = _sc_gather(x.astype(jnp.int32), table)
    return out128[:, :64].reshape(x.shape + (table.shape[1],))
